# baseline (speedup 1.0000x reference)
import functools

import jax
import jax.numpy as jnp
import numpy as np
from jax import lax
from jax.experimental import pallas as pl
from jax.experimental.pallas import tpu as pltpu
from jax.experimental.pallas import tpu_sc as plsc


def _pos_encoding(max_seq_len, d_model):
    pos = np.arange(max_seq_len, dtype=np.float64)[:, None]
    i = np.arange(d_model, dtype=np.float64)[None, :]
    angle = pos / np.power(10000.0, (2.0 * (np.floor(i / 2.0))) / d_model)
    pe = np.where((np.arange(d_model)[None, :] % 2) == 0, np.sin(angle), np.cos(angle))
    return pe.astype(np.float32)


_NW = 32
_CHUNK = 400
_SUBGATHERS = ((0, 128), (128, 128), (256, 128), (384, 16))


@functools.partial(jax.jit, static_argnames=("n_rows", "d"))
def _emb_lookup(tokens_flat, table, pe_rep, *, n_rows, d):
    per_w = n_rows // _NW
    n_chunks = per_w // _CHUNK
    mesh = plsc.VectorSubcoreMesh(core_axis_name="c", subcore_axis_name="s")

    @functools.partial(
        pl.kernel,
        out_type=jax.ShapeDtypeStruct((n_rows, d), jnp.float32),
        mesh=mesh,
        scratch_types=[
            pltpu.VMEM((per_w,), jnp.int32),
            pltpu.VMEM((_CHUNK, d), jnp.float32),
            pltpu.VMEM((_CHUNK, d), jnp.float32),
            pltpu.SemaphoreType.DMA,
        ],
        compiler_params=pltpu.CompilerParams(use_tc_tiling_on_sc=False),
    )
    def k(tokens_hbm, table_hbm, pe_hbm, out_hbm, idx_v, pe_v, rows_v, sem):
        wid = lax.axis_index("s") * 2 + lax.axis_index("c")
        base = wid * per_w
        pltpu.sync_copy(tokens_hbm.at[pl.ds(base, per_w)], idx_v)
        pltpu.sync_copy(pe_hbm, pe_v)

        def chunk_body(c, carry):
            off = c * _CHUNK
            copies = [
                pltpu.async_copy(
                    table_hbm.at[idx_v.at[pl.ds(off + so, n)]],
                    rows_v.at[pl.ds(so, n)],
                    sem,
                )
                for so, n in _SUBGATHERS
            ]
            for cp in copies:
                cp.wait()

            def add_body(r, carry2):
                for j in range(d // 16):
                    sl = pl.ds(j * 16, 16)
                    rows_v[r, sl] = rows_v[r, sl] + pe_v[r, sl]
                return carry2

            lax.fori_loop(0, _CHUNK, add_body, 0, unroll=2)
            pltpu.sync_copy(rows_v, out_hbm.at[pl.ds(base + off, _CHUNK)])
            return carry

        lax.fori_loop(0, n_chunks, chunk_body, 0)

    return k(tokens_flat, table, pe_rep)


def kernel(tokens, table):
    b, s = tokens.shape
    v, d = table.shape
    n_rows = b * s
    pe_rep = jnp.asarray(np.tile(_pos_encoding(s, d), (_CHUNK // s, 1)))
    tokens_flat = tokens.reshape(-1).astype(jnp.int32)
    out = _emb_lookup(tokens_flat, table, pe_rep, n_rows=n_rows, d=d)
    return out.reshape(b, s, d)


# speedup vs baseline: 1.4805x; 1.4805x over previous
import functools

import jax
import jax.numpy as jnp
import numpy as np
from jax import lax
from jax.experimental import pallas as pl
from jax.experimental.pallas import tpu as pltpu
from jax.experimental.pallas import tpu_sc as plsc


def _pos_encoding(max_seq_len, d_model):
    pos = np.arange(max_seq_len, dtype=np.float64)[:, None]
    i = np.arange(d_model, dtype=np.float64)[None, :]
    angle = pos / np.power(10000.0, (2.0 * (np.floor(i / 2.0))) / d_model)
    pe = np.where((np.arange(d_model)[None, :] % 2) == 0, np.sin(angle), np.cos(angle))
    return pe.astype(np.float32)


_NW = 32
_CHUNK = 400
_SUB = ((0, 128), (128, 128), (256, 128), (384, 16))


@functools.partial(jax.jit, static_argnames=("n_rows", "d"))
def _emb_lookup(tokens_flat, table, pe_rep, *, n_rows, d):
    per_w = n_rows // _NW
    n_chunks = per_w // _CHUNK
    mesh = plsc.VectorSubcoreMesh(core_axis_name="c", subcore_axis_name="s")

    @functools.partial(
        pl.kernel,
        out_type=jax.ShapeDtypeStruct((n_rows, d), jnp.float32),
        mesh=mesh,
        scratch_types=[
            pltpu.VMEM((per_w,), jnp.int32),
            pltpu.VMEM((_CHUNK, d), jnp.float32),
            pltpu.VMEM((2, _CHUNK, d), jnp.float32),
            pltpu.SemaphoreType.DMA,
            pltpu.SemaphoreType.DMA,
        ],
        compiler_params=pltpu.CompilerParams(use_tc_tiling_on_sc=False),
    )
    def k(tokens_hbm, table_hbm, pe_hbm, out_hbm, idx_v, pe_v, gbuf, gsem, osem):
        wid = lax.axis_index("s") * 2 + lax.axis_index("c")
        base = wid * per_w
        pltpu.sync_copy(tokens_hbm.at[pl.ds(base, per_w)], idx_v)
        pltpu.sync_copy(pe_hbm, pe_v)

        def gathers(c, b):
            off = c * _CHUNK
            for so, n in _SUB:
                pltpu.async_copy(
                    table_hbm.at[idx_v.at[pl.ds(off + so, n)]],
                    gbuf.at[b, pl.ds(so, n)],
                    gsem,
                )

        def wait_gathers(b):
            for so, n in _SUB:
                pltpu.make_async_copy(
                    table_hbm.at[idx_v.at[pl.ds(so, n)]],
                    gbuf.at[b, pl.ds(so, n)],
                    gsem,
                ).wait()

        def out_copy(c, b):
            pltpu.async_copy(
                gbuf.at[b], out_hbm.at[pl.ds(base + c * _CHUNK, _CHUNK)], osem
            )

        def wait_out(b):
            pltpu.make_async_copy(
                gbuf.at[b], out_hbm.at[pl.ds(base, _CHUNK)], osem
            ).wait()

        def add_pe(b):
            def body(r, carry):
                for j in range(d // 16):
                    sl = pl.ds(j * 16, 16)
                    gbuf[b, r, sl] = gbuf[b, r, sl] + pe_v[r, sl]
                return carry

            lax.fori_loop(0, _CHUNK, body, 0, unroll=2)

        gathers(0, 0)

        def chunk_body(c, carry):
            for b in range(2):
                c2 = 2 * c + b
                nb = 1 - b

                @pl.when(c2 + 1 < n_chunks)
                def _():
                    @pl.when(c2 >= 1)
                    def _():
                        wait_out(nb)

                    gathers(c2 + 1, nb)

                wait_gathers(b)
                add_pe(b)
                out_copy(c2, b)
            return carry

        lax.fori_loop(0, n_chunks // 2, chunk_body, 0)
        wait_out(0)
        wait_out(1)

    return k(tokens_flat, table, pe_rep)


def kernel(tokens, table):
    b, s = tokens.shape
    v, d = table.shape
    n_rows = b * s
    pe_rep = jnp.asarray(np.tile(_pos_encoding(s, d), (_CHUNK // s, 1)))
    tokens_flat = tokens.reshape(-1).astype(jnp.int32)
    out = _emb_lookup(tokens_flat, table, pe_rep, n_rows=n_rows, d=d)
    return out.reshape(b, s, d)
